# Initial kernel scaffold; baseline (speedup 1.0000x reference)
import functools

import jax
import jax.numpy as jnp
from jax import lax
from jax.experimental import pallas as pl
from jax.experimental.pallas import tpu as pltpu
from jax.experimental.pallas import tpu_sc as plsc

S = 2048
D = 1024
E = 8
F = 4096
BM = 256
NT = 24
MP = NT * BM
NW = 32


def _router_body(x_ref, wr_ref, br_ref,
                 tok_ref, posa_ref, posb_ref, wa_ref, wb_ref,
                 gid_ref, valid_ref):
    x = x_ref[...]
    logits = jnp.dot(x, wr_ref[...], preferred_element_type=jnp.float32)
    logits = logits + br_ref[...]
    mx = jnp.max(logits, axis=1, keepdims=True)
    ex = jnp.exp(logits - mx)
    p = ex / jnp.sum(ex, axis=1, keepdims=True)
    ie = lax.broadcasted_iota(jnp.float32, (S, E), 1)
    v1 = jnp.max(p, axis=1, keepdims=True)
    i1 = jnp.min(jnp.where(p == v1, ie, float(E)), axis=1, keepdims=True)
    pm = jnp.where(ie == i1, -1.0, p)
    v2 = jnp.max(pm, axis=1, keepdims=True)
    i2 = jnp.min(jnp.where(pm == v2, ie, float(E)), axis=1, keepdims=True)
    s = v1 + v2
    wa_ref[...] = v1 / s
    wb_ref[...] = v2 / s
    ia = (ie == i1).astype(jnp.float32)
    ib = (ie == i2).astype(jnp.float32)
    cnt_a = jnp.sum(ia, axis=0, keepdims=True)
    cnt = cnt_a + jnp.sum(ib, axis=0, keepdims=True)
    itr = lax.broadcasted_iota(jnp.float32, (S, S), 0)
    itc = lax.broadcasted_iota(jnp.float32, (S, S), 1)
    ltri = (itc < itr).astype(jnp.float32)
    pref = jnp.dot(ltri, jnp.concatenate([ia, ib], axis=1),
                   preferred_element_type=jnp.float32)
    rank_a = jnp.sum(ia * pref[:, :E], axis=1, keepdims=True)
    rank_b = jnp.sum(ib * (pref[:, E:] + cnt_a), axis=1, keepdims=True)
    tl = jnp.floor((cnt + float(BM - 1)) * (1.0 / BM))
    m8r = lax.broadcasted_iota(jnp.float32, (E, E), 0)
    m8c = lax.broadcasted_iota(jnp.float32, (E, E), 1)
    cum = jnp.dot(tl, (m8r <= m8c).astype(jnp.float32),
                  preferred_element_type=jnp.float32)
    off = (cum - tl) * float(BM)
    pos_a = jnp.sum(ia * off, axis=1, keepdims=True) + rank_a
    pos_b = jnp.sum(ib * off, axis=1, keepdims=True) + rank_b
    posa_ref[...] = pos_a.astype(jnp.int32)
    posb_ref[...] = pos_b.astype(jnp.int32)
    total = cum[:, E - 1:E]
    i24 = lax.broadcasted_iota(jnp.float32, (NT, E), 0)
    gidf = jnp.sum((cum <= i24).astype(jnp.float32), axis=1, keepdims=True)
    glast = jnp.sum((cum <= (total - 1.0)).astype(jnp.float32),
                    axis=1, keepdims=True)
    i24c = lax.broadcasted_iota(jnp.float32, (NT, 1), 0)
    validf = (i24c < total)
    gid_ref[...] = jnp.where(validf, gidf, glast).astype(jnp.int32)
    valid_ref[...] = validf.astype(jnp.int32)
    tokcol = lax.broadcasted_iota(jnp.float32, (S, 1), 0)
    tok_pair = jnp.concatenate([tokcol, tokcol], axis=0)
    pos_pair = jnp.concatenate([pos_a, pos_b], axis=0)
    for c in range(MP // 512):
        sl = lax.broadcasted_iota(jnp.float32, (2 * S, 512), 1) + float(c * 512)
        msk = (pos_pair == sl).astype(jnp.float32)
        tok_ref[pl.ds(c, 1), :] = jnp.sum(msk * tok_pair, axis=0,
                                          keepdims=True).astype(jnp.int32)


def _router(x, wr, br2):
    return pl.pallas_call(
        _router_body,
        out_shape=[
            jax.ShapeDtypeStruct((MP // 512, 512), jnp.int32),
            jax.ShapeDtypeStruct((S, 1), jnp.int32),
            jax.ShapeDtypeStruct((S, 1), jnp.int32),
            jax.ShapeDtypeStruct((S, 1), jnp.float32),
            jax.ShapeDtypeStruct((S, 1), jnp.float32),
            jax.ShapeDtypeStruct((NT, 1), jnp.int32),
            jax.ShapeDtypeStruct((NT, 1), jnp.int32),
        ],
    )(x, wr, br2)


def _sc_gather_body(tok_hbm, x_hbm, xs_hbm, idx_v, rows_v, sem):
    wid = lax.axis_index("s") * 2 + lax.axis_index("c")
    rows = MP // NW
    for c in range(rows // 64):
        off = pl.multiple_of(wid * rows + c * 64, 64)
        pltpu.sync_copy(tok_hbm.at[pl.ds(off, 64)], idx_v)
        pltpu.async_copy(x_hbm.at[idx_v], rows_v, sem).wait()
        pltpu.sync_copy(rows_v, xs_hbm.at[pl.ds(off, 64)])


_sc_gather = functools.partial(
    pl.kernel,
    out_type=jax.ShapeDtypeStruct((MP, D), jnp.float32),
    mesh=plsc.VectorSubcoreMesh(core_axis_name="c", subcore_axis_name="s"),
    scratch_types=[
        pltpu.VMEM((64,), jnp.int32),
        pltpu.VMEM((64, D), jnp.float32),
        pltpu.SemaphoreType.DMA,
    ],
)(_sc_gather_body)


def _ffn_body(gid_ref, valid_ref, xs_ref, w1_ref, b1_ref, w2_ref, b2_ref,
              ys_ref):
    m = pl.program_id(0)

    @pl.when(valid_ref[m] != 0)
    def _():
        xb = xs_ref[...].astype(jnp.bfloat16)
        h = jnp.dot(xb, w1_ref[0], preferred_element_type=jnp.float32)
        h = jnp.maximum(h + b1_ref[...], 0.0)
        y = jnp.dot(h.astype(jnp.bfloat16), w2_ref[0],
                    preferred_element_type=jnp.float32)
        ys_ref[...] = y + b2_ref[...]


def _ffn(gid, valid, xs, w1b, b1, w2b, b2):
    grid_spec = pltpu.PrefetchScalarGridSpec(
        num_scalar_prefetch=2,
        grid=(NT,),
        in_specs=[
            pl.BlockSpec((BM, D), lambda m, g, v: (m, 0)),
            pl.BlockSpec((1, D, F), lambda m, g, v: (g[m], 0, 0)),
            pl.BlockSpec((1, F), lambda m, g, v: (g[m], 0)),
            pl.BlockSpec((1, F, D), lambda m, g, v: (g[m], 0, 0)),
            pl.BlockSpec((1, D), lambda m, g, v: (g[m], 0)),
        ],
        out_specs=pl.BlockSpec((BM, D), lambda m, g, v: (m, 0)),
    )
    return pl.pallas_call(
        _ffn_body,
        grid_spec=grid_spec,
        out_shape=jax.ShapeDtypeStruct((MP, D), jnp.float32),
        compiler_params=pltpu.CompilerParams(
            dimension_semantics=("arbitrary",)),
    )(gid, valid, xs, w1b, b1, w2b, b2)


def _sc_pick_body(ys_hbm, posa_hbm, posb_hbm, ya_hbm, yb_hbm, idx_v, buf_v,
                  sem):
    wid = lax.axis_index("s") * 2 + lax.axis_index("c")
    base = pl.multiple_of(wid * (S // NW), S // NW)
    for pos_hbm, y_hbm in ((posa_hbm, ya_hbm), (posb_hbm, yb_hbm)):
        pltpu.sync_copy(pos_hbm.at[pl.ds(base, S // NW)], idx_v)
        pltpu.async_copy(ys_hbm.at[idx_v], buf_v, sem).wait()
        pltpu.sync_copy(buf_v, y_hbm.at[pl.ds(base, S // NW)])


_sc_pick = functools.partial(
    pl.kernel,
    out_type=[
        jax.ShapeDtypeStruct((S, D), jnp.float32),
        jax.ShapeDtypeStruct((S, D), jnp.float32),
    ],
    mesh=plsc.VectorSubcoreMesh(core_axis_name="c", subcore_axis_name="s"),
    scratch_types=[
        pltpu.VMEM((S // NW,), jnp.int32),
        pltpu.VMEM((S // NW, D), jnp.float32),
        pltpu.SemaphoreType.DMA,
    ],
)(_sc_pick_body)


def _combine_body(wa_ref, wb_ref, ya_ref, yb_ref, out_ref):
    out_ref[...] = wa_ref[...] * ya_ref[...] + wb_ref[...] * yb_ref[...]


def _combine(wa, wb, ya, yb):
    return pl.pallas_call(
        _combine_body,
        out_shape=jax.ShapeDtypeStruct((S, D), jnp.float32),
    )(wa, wb, ya, yb)


def kernel(input_emb, Wr, br, W1, b1, W2, b2):
    x = input_emb.reshape(S, D)
    tok12, posa, posb, wa, wb, gid, valid = _router(x, Wr, br.reshape(1, E))
    tok = tok12.reshape(MP)
    xs = _sc_gather(tok, x)
    w1b = W1.astype(jnp.bfloat16)
    w2b = W2.astype(jnp.bfloat16)
    ys = _ffn(gid.reshape(NT), valid.reshape(NT), xs, w1b, b1, w2b, b2)
    ya, yb = _sc_pick(ys, posa.reshape(S), posb.reshape(S))
    out = _combine(wa, wb, ya, yb)
    return out.reshape(1, S, D)


# speedup vs baseline: 1.3162x; 1.3162x over previous
import jax
import jax.numpy as jnp
from jax import lax
from jax.experimental import pallas as pl
from jax.experimental.pallas import tpu as pltpu
from jax.experimental.pallas import tpu_sc as plsc

S = 2048
D = 1024
E = 8
F = 4096
BM = 256
NT = 23
MP = NT * BM
NW = 32


def _fiota(shape, dim):
    return lax.broadcasted_iota(jnp.int32, shape, dim).astype(jnp.float32)


def _router_body(x_ref, wr_ref, br_ref,
                 posa_ref, posb_ref, wa_ref, wb_ref,
                 gid_ref, valid_ref):
    x = x_ref[...]
    logits = jnp.dot(x, wr_ref[...], preferred_element_type=jnp.float32)
    logits = logits + br_ref[...]
    mx = jnp.max(logits, axis=1, keepdims=True)
    ex = jnp.exp(logits - mx)
    p = ex / jnp.sum(ex, axis=1, keepdims=True)
    ie = _fiota((S, E), 1)
    v1 = jnp.max(p, axis=1, keepdims=True)
    i1 = jnp.min(jnp.where(p == v1, ie, float(E)), axis=1, keepdims=True)
    pm = jnp.where(ie == i1, -1.0, p)
    v2 = jnp.max(pm, axis=1, keepdims=True)
    i2 = jnp.min(jnp.where(pm == v2, ie, float(E)), axis=1, keepdims=True)
    s = v1 + v2
    wa_ref[...] = v1 / s
    wb_ref[...] = v2 / s
    ia = (ie == i1).astype(jnp.float32)
    ib = (ie == i2).astype(jnp.float32)
    cnt_a = jnp.sum(ia, axis=0, keepdims=True)
    cnt = cnt_a + jnp.sum(ib, axis=0, keepdims=True)
    l128r = _fiota((128, 128), 0)
    l128c = _fiota((128, 128), 1)
    l128 = (l128c < l128r).astype(jnp.float32)
    iab = jnp.concatenate([ia, ib], axis=1)
    base = jnp.zeros((1, 2 * E), jnp.float32)
    parts = []
    for b in range(S // 128):
        seg = iab[b * 128:(b + 1) * 128, :]
        parts.append(jnp.dot(l128, seg, preferred_element_type=jnp.float32)
                     + base)
        base = base + jnp.sum(seg, axis=0, keepdims=True)
    pref = jnp.concatenate(parts, axis=0)
    rank_a = jnp.sum(ia * pref[:, :E], axis=1, keepdims=True)
    rank_b = jnp.sum(ib * (pref[:, E:] + cnt_a), axis=1, keepdims=True)
    tl = jnp.floor((cnt + float(BM - 1)) * (1.0 / BM))
    m8r = _fiota((E, E), 0)
    m8c = _fiota((E, E), 1)
    cum = jnp.dot(tl, (m8r <= m8c).astype(jnp.float32),
                  preferred_element_type=jnp.float32)
    off = (cum - tl) * float(BM)
    pos_a = jnp.sum(ia * off, axis=1, keepdims=True) + rank_a
    pos_b = jnp.sum(ib * off, axis=1, keepdims=True) + rank_b
    posa_ref[...] = pos_a.astype(jnp.int32)
    posb_ref[...] = pos_b.astype(jnp.int32)
    total = cum[:, E - 1:E]
    i24 = _fiota((NT, E), 0)
    gidf = jnp.sum((cum <= i24).astype(jnp.float32), axis=1, keepdims=True)
    glast = jnp.sum((cum <= (total - 1.0)).astype(jnp.float32),
                    axis=1, keepdims=True)
    i24c = _fiota((NT, 1), 0)
    validf = (i24c < total)
    gid_ref[...] = jnp.where(validf, gidf, glast).astype(jnp.int32)
    valid_ref[...] = validf.astype(jnp.int32)


def _router(x, wr, br2):
    return pl.pallas_call(
        _router_body,
        out_shape=[
            jax.ShapeDtypeStruct((S, 1), jnp.int32),
            jax.ShapeDtypeStruct((S, 1), jnp.int32),
            jax.ShapeDtypeStruct((S, 1), jnp.float32),
            jax.ShapeDtypeStruct((S, 1), jnp.float32),
            jax.ShapeDtypeStruct((NT, 1), jnp.int32),
            jax.ShapeDtypeStruct((NT, 1), jnp.int32),
        ],
    )(x, wr, br2)


def _sc_scatter_body(x_hbm, posa_hbm, posb_hbm, xs_hbm, idxa_v, idxb_v, buf_v,
                     sem):
    wid = lax.axis_index("s") * 2 + lax.axis_index("c")
    n = S // NW
    base = pl.multiple_of(wid * n, n)
    c0 = pltpu.async_copy(posa_hbm.at[pl.ds(base, n)], idxa_v, sem)
    c1 = pltpu.async_copy(posb_hbm.at[pl.ds(base, n)], idxb_v, sem)
    c2 = pltpu.async_copy(x_hbm.at[pl.ds(base, n)], buf_v, sem)
    c0.wait()
    c1.wait()
    c2.wait()
    ca = pltpu.async_copy(buf_v, xs_hbm.at[idxa_v], sem)
    cb = pltpu.async_copy(buf_v, xs_hbm.at[idxb_v], sem)
    ca.wait()
    cb.wait()


def _sc_scatter(x, posa, posb):
    k = pl.kernel(
        _sc_scatter_body,
        out_type=jax.ShapeDtypeStruct((MP, D), jnp.float32),
        mesh=plsc.VectorSubcoreMesh(core_axis_name="c", subcore_axis_name="s"),
        scratch_types=[
            pltpu.VMEM((S // NW,), jnp.int32),
            pltpu.VMEM((S // NW,), jnp.int32),
            pltpu.VMEM((S // NW, D), jnp.float32),
            pltpu.SemaphoreType.DMA,
        ],
    )
    return k(x, posa, posb)


def _ffn1_body(gid_ref, valid_ref, xs_ref, w1_ref, b1_ref, h_ref):
    m = pl.program_id(0)

    @pl.when(valid_ref[m] != 0)
    def _():
        t = jnp.dot(xs_ref[...], w1_ref[0], preferred_element_type=jnp.float32)
        h_ref[...] = jnp.maximum(t + b1_ref[0], 0.0).astype(jnp.bfloat16)


def _ffn2_body(gid_ref, valid_ref, h_ref, w2_ref, b2_ref, ys_ref):
    m = pl.program_id(0)

    @pl.when(valid_ref[m] != 0)
    def _():
        h = h_ref[...].astype(jnp.float32)
        y = jnp.dot(h, w2_ref[0], preferred_element_type=jnp.float32)
        ys_ref[...] = y + b2_ref[0]


def _ffn(gid, valid, xs, w1, b1, w2, b2):
    spec1 = pltpu.PrefetchScalarGridSpec(
        num_scalar_prefetch=2,
        grid=(NT,),
        in_specs=[
            pl.BlockSpec((BM, D), lambda m, g, v: (m, 0)),
            pl.BlockSpec((1, D, F), lambda m, g, v: (g[m], 0, 0)),
            pl.BlockSpec((1, 1, F), lambda m, g, v: (g[m], 0, 0)),
        ],
        out_specs=pl.BlockSpec((BM, F), lambda m, g, v: (m, 0)),
    )
    h = pl.pallas_call(
        _ffn1_body,
        grid_spec=spec1,
        out_shape=jax.ShapeDtypeStruct((MP, F), jnp.bfloat16),
        compiler_params=pltpu.CompilerParams(
            dimension_semantics=("arbitrary",)),
    )(gid, valid, xs, w1, b1.reshape(E, 1, F))
    spec2 = pltpu.PrefetchScalarGridSpec(
        num_scalar_prefetch=2,
        grid=(NT,),
        in_specs=[
            pl.BlockSpec((BM, F), lambda m, g, v: (m, 0)),
            pl.BlockSpec((1, F, D), lambda m, g, v: (g[m], 0, 0)),
            pl.BlockSpec((1, 1, D), lambda m, g, v: (g[m], 0, 0)),
        ],
        out_specs=pl.BlockSpec((BM, D), lambda m, g, v: (m, 0)),
    )
    return pl.pallas_call(
        _ffn2_body,
        grid_spec=spec2,
        out_shape=jax.ShapeDtypeStruct((MP, D), jnp.float32),
        compiler_params=pltpu.CompilerParams(
            dimension_semantics=("arbitrary",)),
    )(gid, valid, h, w2, b2.reshape(E, 1, D))


def _sc_pick_body(ys_hbm, posa_hbm, posb_hbm, ya_hbm, yb_hbm, idx_v, buf_v,
                  sem):
    wid = lax.axis_index("s") * 2 + lax.axis_index("c")
    n = S // NW
    base = pl.multiple_of(wid * n, n)
    for pos_hbm, y_hbm in ((posa_hbm, ya_hbm), (posb_hbm, yb_hbm)):
        pltpu.sync_copy(pos_hbm.at[pl.ds(base, n)], idx_v)
        pltpu.async_copy(ys_hbm.at[idx_v], buf_v, sem).wait()
        pltpu.sync_copy(buf_v, y_hbm.at[pl.ds(base, n)])


def _sc_pick(ys, posa, posb):
    k = pl.kernel(
        _sc_pick_body,
        out_type=[
            jax.ShapeDtypeStruct((S, D), jnp.float32),
            jax.ShapeDtypeStruct((S, D), jnp.float32),
        ],
        mesh=plsc.VectorSubcoreMesh(core_axis_name="c", subcore_axis_name="s"),
        scratch_types=[
            pltpu.VMEM((S // NW,), jnp.int32),
            pltpu.VMEM((S // NW, D), jnp.float32),
            pltpu.SemaphoreType.DMA,
        ],
    )
    return k(ys, posa, posb)


def _combine_body(wa_ref, wb_ref, ya_ref, yb_ref, out_ref):
    out_ref[...] = wa_ref[...] * ya_ref[...] + wb_ref[...] * yb_ref[...]


def _combine(wa, wb, ya, yb):
    return pl.pallas_call(
        _combine_body,
        out_shape=jax.ShapeDtypeStruct((S, D), jnp.float32),
    )(wa, wb, ya, yb)

def kernel(input_emb, Wr, br, W1, b1, W2, b2):
    x = input_emb.reshape(S, D)
    posa, posb, wa, wb, gid, valid = _router(x, Wr, br.reshape(1, E))
    xs = _sc_scatter(x, posa.reshape(S), posb.reshape(S))
    ys = _ffn(gid.reshape(NT), valid.reshape(NT), xs, W1, b1, W2, b2)
    ya, yb = _sc_pick(ys, posa.reshape(S), posb.reshape(S))
    out = _combine(wa, wb, ya, yb)
    return out.reshape(1, S, D)
